# baseline (speedup 1.0000x reference)
import functools

import jax
import jax.numpy as jnp
from jax import lax
from jax.experimental import pallas as pl
from jax.experimental.pallas import tpu as pltpu
from jax.experimental.pallas import tpu_sc as plsc

_NC = 2
_NS = 16
_NW = _NC * _NS
_W = 128
_NBUF = 8
_LOOKAHEAD = 4
_CHUNK = 32


def kernel(datasets, perm):
    B, N, D = datasets.shape
    cpb = N // _W
    perm2d = perm.astype(jnp.int32).reshape(cpb, _W)
    nb_per_w = B // _NW
    m = nb_per_w * cpb

    mesh = plsc.VectorSubcoreMesh(core_axis_name="c", subcore_axis_name="s")

    @functools.partial(
        pl.kernel,
        out_type=jax.ShapeDtypeStruct((B, N, D), datasets.dtype),
        mesh=mesh,
        scratch_types=[
            pltpu.VMEM((cpb, _W), jnp.int32),
            pltpu.VMEM((_NBUF, _W, D), jnp.float32),
            pltpu.SemaphoreType.DMA((_NBUF,)),
            pltpu.SemaphoreType.DMA((_NBUF,)),
        ],
        compiler_params=pltpu.CompilerParams(use_tc_tiling_on_sc=False),
    )
    def _k(data_hbm, perm_hbm, out_hbm, perm_v, rows_v, gsem, wsem):
        wid = lax.axis_index("s") * _NC + lax.axis_index("c")
        pltpu.sync_copy(perm_hbm, perm_v)
        b0 = wid * nb_per_w

        def g_copy(c, s):
            b = b0 + c // cpb
            j = c % cpb
            return pltpu.async_copy(
                data_hbm.at[b].at[perm_v.at[j]], rows_v.at[s], gsem.at[s])

        def w_copy(c, s):
            b = b0 + c // cpb
            j = c % cpb
            return pltpu.async_copy(
                rows_v.at[s], out_hbm.at[b].at[pl.ds(j * _W, _W)],
                wsem.at[s])

        @pl.loop(0, m // _CHUNK)
        def _chunk(q):
            c0 = q * _CHUNK
            gh = [None] * _CHUNK
            wh = [None] * _CHUNK
            for s in range(_LOOKAHEAD):
                gh[s] = g_copy(c0 + s, s)
            for p in range(_CHUNK):
                gh[p].wait()
                wh[p] = w_copy(c0 + p, p % _NBUF)
                pn = p + _LOOKAHEAD
                if pn < _CHUNK:
                    if p >= _LOOKAHEAD:
                        wh[p - _LOOKAHEAD].wait()
                    gh[pn] = g_copy(c0 + pn, pn % _NBUF)
            for p in range(_CHUNK - _NBUF, _CHUNK):
                wh[p].wait()

    return _k(datasets, perm2d)
